# baseline (speedup 1.0000x reference)
import functools

import jax
import jax.numpy as jnp
from jax import lax
from jax.experimental import pallas as pl
from jax.experimental.pallas import tpu as pltpu
from jax.experimental.pallas import tpu_sc as plsc

_N, _D, _K, _KP = 10000, 128, 30, 32
_NW = 32
_LANE = 128
_RPT = 320
_NPAD = 10240


def _pass_a(x_ref, aw_ref, ab_ref, pw_ref, pb_ref, s32_ref, z_ref):
    i = pl.program_id(0)
    xb = x_ref[...]
    logits = jnp.dot(xb, aw_ref[...], preferred_element_type=jnp.float32)
    logits = logits + ab_ref[...]
    m = jnp.max(logits, axis=-1, keepdims=True)
    e = jnp.exp(logits - m)
    s = e / jnp.sum(e, axis=-1, keepdims=True)
    colid = lax.broadcasted_iota(jnp.int32, s.shape, 1)
    s32 = jnp.where(colid == _K, 1.0, s)
    s32_ref[...] = s32.astype(jnp.bfloat16)
    xp = jnp.dot(xb, pw_ref[...], preferred_element_type=jnp.float32)
    xp = xp + pb_ref[...]
    zp = lax.dot_general(s32[:, :_KP], xp, (((0,), (0,)), ((), ())),
                         preferred_element_type=jnp.float32)

    @pl.when(i == 0)
    def _():
        z_ref[...] = zp

    @pl.when(i > 0)
    def _():
        z_ref[...] += zp


def _make_seg_sum(ngrp):
    mesh = plsc.VectorSubcoreMesh(core_axis_name="c", subcore_axis_name="s")

    depth = 8
    nbuf = 2 * depth
    q, rem = divmod(ngrp, _NW)
    assert q >= nbuf

    @functools.partial(
        pl.kernel,
        out_type=jax.ShapeDtypeStruct((2, _NPAD, 128), jnp.bfloat16),
        mesh=mesh,
        scratch_types=[
            pltpu.VMEM((q + 1, _LANE), jnp.int32),
            pltpu.VMEM((q + 1, _LANE), jnp.int32),
            pltpu.VMEM((nbuf, _LANE, _KP), jnp.bfloat16),
            pltpu.VMEM_SHARED((_NPAD, _KP), jnp.bfloat16),
            pltpu.VMEM_SHARED((_N, _KP), jnp.bfloat16),
            pltpu.SemaphoreType.DMA,
            pltpu.SemaphoreType.DMA,
        ],
        compiler_params=pltpu.CompilerParams(use_tc_tiling_on_sc=False),
    )
    def seg_sum(s32_hbm, e3_hbm, zeros_hbm, out_hbm,
                colv, rowv, rowsv, shared, table, gsem, ssem):
        cid = lax.axis_index("c")
        sid = lax.axis_index("s")
        wid = sid * 2 + cid
        base = q * wid + jnp.minimum(wid, rem)
        nch = q + jnp.where(wid < rem, 1, 0)
        pltpu.sync_copy(zeros_hbm.at[pl.ds(sid * 2 * _RPT, 2 * _RPT)],
                        shared.at[pl.ds(sid * 2 * _RPT, 2 * _RPT)])
        pltpu.sync_copy(
            s32_hbm.at[pl.ds(sid * (_N // 16), _N // 16), pl.ds(0, _KP)],
            table.at[pl.ds(sid * (_N // 16), _N // 16)])
        pltpu.sync_copy(e3_hbm.at[pl.ds(base, q), 1], colv.at[pl.ds(0, q)])
        pltpu.sync_copy(e3_hbm.at[pl.ds(base, q), 0], rowv.at[pl.ds(0, q)])

        @pl.when(wid < rem)
        def _():
            pltpu.sync_copy(e3_hbm.at[pl.ds(base + q, 1), 1],
                            colv.at[pl.ds(q, 1)])
            pltpu.sync_copy(e3_hbm.at[pl.ds(base + q, 1), 0],
                            rowv.at[pl.ds(q, 1)])

        plsc.subcore_barrier()

        def buf(j):
            return rowsv.at[lax.rem(j, nbuf)]

        def gstart(j):
            pltpu.async_copy(table.at[colv.at[j]], buf(j), gsem)

        def gwait(j):
            pltpu.make_async_copy(table.at[colv.at[j]], buf(j), gsem).wait()

        def sstart(j):
            pltpu.async_copy(buf(j), shared.at[rowv.at[j]], ssem, add=True)

        def swait(j):
            pltpu.make_async_copy(buf(j), shared.at[rowv.at[j]], ssem).wait()

        for j in range(depth):
            gstart(j)
        for j in range(depth):
            gstart(j + depth)
            gwait(j)
            sstart(j)

        def body(j, carry):
            swait(j - depth)
            gstart(j + depth)
            gwait(j)
            sstart(j)
            return carry

        lax.fori_loop(depth, nch - depth, body, 0)

        def tail(j, carry):
            swait(j - depth)
            gwait(j)
            sstart(j)
            return carry

        lax.fori_loop(nch - depth, nch, tail, 0)

        def drain(j, carry):
            swait(j)
            return carry

        lax.fori_loop(nch - depth, nch, drain, 0)
        plsc.subcore_barrier()
        pltpu.sync_copy(
            shared.at[pl.ds(sid * 2 * _RPT, 2 * _RPT)],
            out_hbm.at[cid, pl.ds(sid * 2 * _RPT, 2 * _RPT), pl.ds(0, _KP)])

    return seg_sum


def _explain(xk, a_mask, w_top, w_bot, m1_b, m2_w, m2_b, lin_w, lin_b):
    u = jnp.dot(xk, w_top, preferred_element_type=jnp.float32)
    v = jnp.dot(xk, w_bot, preferred_element_type=jnp.float32)
    h = jnp.maximum(u[:, None, :] + v[None, :, :] + m1_b, 0.0)
    logit = jnp.sum(h * m2_w, axis=-1) + m2_b[...]
    mask = (1.0 / (1.0 + jnp.exp(-logit))) * a_mask
    wsum = jnp.sum(mask, axis=1, keepdims=True)
    w = mask / (wsum + 1e-09)
    agg = jnp.dot(w, xk, preferred_element_type=jnp.float32)
    deg = jnp.sum(a_mask, axis=1, keepdims=True)
    emb = jnp.where(deg > 0, 0.5 * xk + 0.5 * agg, xk)
    out = jnp.dot(emb, lin_w, preferred_element_type=jnp.float32) + lin_b
    return jnp.maximum(out, 0.0)


def _pass_c(s32_ref, adj_ref, z_ref,
            e1w_top, e1w_bot, e1m1b, e1m2w, e1m2b, e1lw, e1lb,
            e2w_top, e2w_bot, e2m1b, e2m2w, e2m2b, e2lw, e2lb,
            lw_ref, lb_ref,
            out_ref, mc_ref, or_ref,
            acc_ref, ss_ref):
    i = pl.program_id(0)
    n_blk = pl.num_programs(0)
    s32 = s32_ref[...][:, :_KP].astype(jnp.float32)
    adj = (adj_ref[0, :, :_KP].astype(jnp.float32)
           + adj_ref[1, :, :_KP].astype(jnp.float32))
    part = lax.dot_general(s32, adj, (((0,), (0,)), ((), ())),
                           preferred_element_type=jnp.float32)
    ssp = lax.dot_general(s32, s32, (((0,), (0,)), ((), ())),
                          preferred_element_type=jnp.float32)

    @pl.when(i == 0)
    def _():
        acc_ref[...] = part
        ss_ref[...] = ssp

    @pl.when(i > 0)
    def _():
        acc_ref[...] += part
        ss_ref[...] += ssp

    @pl.when(i == n_blk - 1)
    def _():
        acc = acc_ref[...]
        adj_new = acc[:_K, :_K]
        r = lax.broadcasted_iota(jnp.int32, (_K, _K), 0)
        c = lax.broadcasted_iota(jnp.int32, (_K, _K), 1)
        eye = (r == c).astype(jnp.float32)
        cut = jnp.sum(adj_new * eye, keepdims=True)
        vol = jnp.sum(acc[:_K, _K:_K + 1], keepdims=True)
        mc_ref[...] = -cut / (vol + 1e-09)
        ss = ss_ref[:_K, :_K]
        or_ref[...] = jnp.sqrt(jnp.sum((ss - eye) ** 2, keepdims=True))
        a_mask = (adj_new > 0).astype(jnp.float32)
        zk = z_ref[:_K, :]
        x1 = _explain(zk, a_mask, e1w_top[...], e1w_bot[...], e1m1b[...],
                      e1m2w[...], e1m2b[...], e1lw[...], e1lb[...])
        x2 = _explain(x1, a_mask, e2w_top[...], e2w_bot[...], e2m1b[...],
                      e2m2w[...], e2m2b[...], e2lw[...], e2lb[...])
        out_ref[...] = (jnp.dot(x2, lw_ref[...],
                                preferred_element_type=jnp.float32)
                        + lb_ref[...])


def kernel(x, edge_index, assign_W, assign_b, proj_W, proj_b,
           e1_m1_W, e1_m1_b, e1_m2_W, e1_m2_b, e1_lin_W, e1_lin_b,
           e2_m1_W, e2_m1_b, e2_m2_W, e2_m2_b, e2_lin_W, e2_lin_b,
           lin_W, lin_b):
    n, d = x.shape
    e = edge_index.shape[1]
    blk = 1000
    n_blk = n // blk

    aw = jnp.pad(assign_W, ((0, 0), (0, 128 - _K)))
    ab = jnp.pad(assign_b, (0, 128 - _K), constant_values=-1e9).reshape(1, 128)
    pb = proj_b.reshape(1, d)

    s32, z32 = pl.pallas_call(
        _pass_a,
        grid=(n_blk,),
        in_specs=[
            pl.BlockSpec((blk, d), lambda i: (i, 0)),
            pl.BlockSpec((d, 128), lambda i: (0, 0)),
            pl.BlockSpec((1, 128), lambda i: (0, 0)),
            pl.BlockSpec((d, d), lambda i: (0, 0)),
            pl.BlockSpec((1, d), lambda i: (0, 0)),
        ],
        out_specs=[
            pl.BlockSpec((blk, 128), lambda i: (i, 0)),
            pl.BlockSpec((_KP, d), lambda i: (0, 0)),
        ],
        out_shape=[
            jax.ShapeDtypeStruct((n, 128), jnp.bfloat16),
            jax.ShapeDtypeStruct((_KP, d), jnp.float32),
        ],
    )(x, aw, ab, proj_W, pb)

    ngrp = e // _LANE
    assert e % _LANE == 0
    e3 = edge_index.reshape(2, ngrp, _LANE).transpose(1, 0, 2)
    zeros = jnp.zeros((_NPAD, _KP), jnp.bfloat16)

    parts = _make_seg_sum(ngrp)(s32, e3, zeros)

    e1m2w = e1_m2_W.reshape(1, -1)
    e2m2w = e2_m2_W.reshape(1, -1)

    def whole(shape):
        return pl.BlockSpec(shape, lambda i: tuple(0 for _ in shape))

    o = lin_W.shape[1]
    out, mc, ortho = pl.pallas_call(
        _pass_c,
        grid=(n_blk,),
        in_specs=[
            pl.BlockSpec((blk, 128), lambda i: (i, 0)),
            pl.BlockSpec((2, blk, 128), lambda i: (0, i, 0)),
            whole((_KP, d)),
            whole((d, d)), whole((d, d)), whole((1, d)),
            whole((1, d)), whole((1, 1)), whole((d, d)), whole((1, d)),
            whole((d, d)), whole((d, d)), whole((1, d)),
            whole((1, d)), whole((1, 1)), whole((d, d)), whole((1, d)),
            whole((d, o)), whole((1, o)),
        ],
        out_specs=[
            whole((_K, o)),
            whole((1, 1)),
            whole((1, 1)),
        ],
        out_shape=[
            jax.ShapeDtypeStruct((_K, o), jnp.float32),
            jax.ShapeDtypeStruct((1, 1), jnp.float32),
            jax.ShapeDtypeStruct((1, 1), jnp.float32),
        ],
        scratch_shapes=[
            pltpu.VMEM((_KP, _KP), jnp.float32),
            pltpu.VMEM((_KP, _KP), jnp.float32),
        ],
    )(s32, parts, z32,
      e1_m1_W[:d], e1_m1_W[d:], e1_m1_b.reshape(1, d),
      e1m2w, e1_m2_b.reshape(1, 1), e1_lin_W, e1_lin_b.reshape(1, -1),
      e2_m1_W[:d], e2_m1_W[d:], e2_m1_b.reshape(1, d),
      e2m2w, e2_m2_b.reshape(1, 1), e2_lin_W, e2_lin_b.reshape(1, -1),
      lin_W, lin_b.reshape(1, -1))

    return (out, mc.reshape(()), ortho.reshape(()), z32[:_K],
            s32[:, :_K].astype(jnp.float32))


# speedup vs baseline: 1.2416x; 1.2416x over previous
import functools

import jax
import jax.numpy as jnp
from jax import lax
from jax.experimental import pallas as pl
from jax.experimental.pallas import tpu as pltpu
from jax.experimental.pallas import tpu_sc as plsc

_N, _D, _K, _KP = 10000, 128, 30, 32
_NW = 32
_LANE = 128
_RPT = 320
_NPAD = 10240


def _pass_a(x_ref, aw_ref, ab_ref, pw_ref, pb_ref, s32_ref, z_ref):
    i = pl.program_id(0)
    xb = x_ref[...]
    logits = jnp.dot(xb, aw_ref[...], preferred_element_type=jnp.float32)
    logits = logits + ab_ref[...]
    m = jnp.max(logits, axis=-1, keepdims=True)
    e = jnp.exp(logits - m)
    s = e / jnp.sum(e, axis=-1, keepdims=True)
    colid = lax.broadcasted_iota(jnp.int32, s.shape, 1)
    s32 = jnp.where(colid == _K, 1.0, s)
    s32_ref[...] = s32
    xp = jnp.dot(xb, pw_ref[...], preferred_element_type=jnp.float32)
    xp = xp + pb_ref[...]
    zp = lax.dot_general(s32[:, :_KP], xp, (((0,), (0,)), ((), ())),
                         preferred_element_type=jnp.float32)

    @pl.when(i == 0)
    def _():
        z_ref[...] = zp

    @pl.when(i > 0)
    def _():
        z_ref[...] += zp


def _make_seg_sum(ngrp):
    mesh = plsc.VectorSubcoreMesh(core_axis_name="c", subcore_axis_name="s")

    depth = 8
    nbuf = 2 * depth
    q, rem = divmod(ngrp, _NW)
    assert q >= nbuf

    @functools.partial(
        pl.kernel,
        out_type=jax.ShapeDtypeStruct((2, _NPAD, 128), jnp.float32),
        mesh=mesh,
        scratch_types=[
            pltpu.VMEM((q + 1, _LANE), jnp.int32),
            pltpu.VMEM((q + 1, _LANE), jnp.int32),
            pltpu.VMEM((nbuf, _LANE, _KP), jnp.float32),
            pltpu.VMEM_SHARED((_NPAD, _KP), jnp.float32),
            pltpu.VMEM_SHARED((_N, _KP), jnp.float32),
            pltpu.SemaphoreType.DMA,
            pltpu.SemaphoreType.DMA,
        ],
        compiler_params=pltpu.CompilerParams(use_tc_tiling_on_sc=False),
    )
    def seg_sum(s32_hbm, e3_hbm, zeros_hbm, out_hbm,
                colv, rowv, rowsv, shared, table, gsem, ssem):
        cid = lax.axis_index("c")
        sid = lax.axis_index("s")
        wid = sid * 2 + cid
        base = q * wid + jnp.minimum(wid, rem)
        nch = q + jnp.where(wid < rem, 1, 0)
        pltpu.sync_copy(zeros_hbm.at[pl.ds(sid * 2 * _RPT, 2 * _RPT)],
                        shared.at[pl.ds(sid * 2 * _RPT, 2 * _RPT)])
        pltpu.sync_copy(
            s32_hbm.at[pl.ds(sid * (_N // 16), _N // 16), pl.ds(0, _KP)],
            table.at[pl.ds(sid * (_N // 16), _N // 16)])
        pltpu.sync_copy(e3_hbm.at[pl.ds(base, q), 1], colv.at[pl.ds(0, q)])
        pltpu.sync_copy(e3_hbm.at[pl.ds(base, q), 0], rowv.at[pl.ds(0, q)])

        @pl.when(wid < rem)
        def _():
            pltpu.sync_copy(e3_hbm.at[pl.ds(base + q, 1), 1],
                            colv.at[pl.ds(q, 1)])
            pltpu.sync_copy(e3_hbm.at[pl.ds(base + q, 1), 0],
                            rowv.at[pl.ds(q, 1)])

        plsc.subcore_barrier()

        def buf(j):
            return rowsv.at[lax.rem(j, nbuf)]

        def gstart(j):
            pltpu.async_copy(table.at[colv.at[j]], buf(j), gsem)

        def gwait(j):
            pltpu.make_async_copy(table.at[colv.at[j]], buf(j), gsem).wait()

        def sstart(j):
            pltpu.async_copy(buf(j), shared.at[rowv.at[j]], ssem, add=True)

        def swait(j):
            pltpu.make_async_copy(buf(j), shared.at[rowv.at[j]], ssem).wait()

        for j in range(depth):
            gstart(j)
        for j in range(depth):
            gstart(j + depth)
            gwait(j)
            sstart(j)

        def body(j, carry):
            swait(j - depth)
            gstart(j + depth)
            gwait(j)
            sstart(j)
            return carry

        lax.fori_loop(depth, nch - depth, body, 0)

        def tail(j, carry):
            swait(j - depth)
            gwait(j)
            sstart(j)
            return carry

        lax.fori_loop(nch - depth, nch, tail, 0)

        def drain(j, carry):
            swait(j)
            return carry

        lax.fori_loop(nch - depth, nch, drain, 0)
        plsc.subcore_barrier()
        pltpu.sync_copy(
            shared.at[pl.ds(sid * 2 * _RPT, 2 * _RPT)],
            out_hbm.at[cid, pl.ds(sid * 2 * _RPT, 2 * _RPT), pl.ds(0, _KP)])

    return seg_sum


def _explain(xk, a_mask, w_top, w_bot, m1_b, m2_w, m2_b, lin_w, lin_b):
    u = jnp.dot(xk, w_top, preferred_element_type=jnp.float32)
    v = jnp.dot(xk, w_bot, preferred_element_type=jnp.float32)
    h = jnp.maximum(u[:, None, :] + v[None, :, :] + m1_b, 0.0)
    logit = jnp.sum(h * m2_w, axis=-1) + m2_b[...]
    mask = (1.0 / (1.0 + jnp.exp(-logit))) * a_mask
    wsum = jnp.sum(mask, axis=1, keepdims=True)
    w = mask / (wsum + 1e-09)
    agg = jnp.dot(w, xk, preferred_element_type=jnp.float32)
    deg = jnp.sum(a_mask, axis=1, keepdims=True)
    emb = jnp.where(deg > 0, 0.5 * xk + 0.5 * agg, xk)
    out = jnp.dot(emb, lin_w, preferred_element_type=jnp.float32) + lin_b
    return jnp.maximum(out, 0.0)


def _pass_c(s32_ref, adj_ref, z_ref,
            e1w_top, e1w_bot, e1m1b, e1m2w, e1m2b, e1lw, e1lb,
            e2w_top, e2w_bot, e2m1b, e2m2w, e2m2b, e2lw, e2lb,
            lw_ref, lb_ref,
            out_ref, mc_ref, or_ref,
            acc_ref, ss_ref):
    i = pl.program_id(0)
    n_blk = pl.num_programs(0)
    s32 = s32_ref[...][:, :_KP]
    adj = adj_ref[0, :, :_KP] + adj_ref[1, :, :_KP]
    part = lax.dot_general(s32, adj, (((0,), (0,)), ((), ())),
                           preferred_element_type=jnp.float32)
    ssp = lax.dot_general(s32, s32, (((0,), (0,)), ((), ())),
                          preferred_element_type=jnp.float32)

    @pl.when(i == 0)
    def _():
        acc_ref[...] = part
        ss_ref[...] = ssp

    @pl.when(i > 0)
    def _():
        acc_ref[...] += part
        ss_ref[...] += ssp

    @pl.when(i == n_blk - 1)
    def _():
        acc = acc_ref[...]
        adj_new = acc[:_K, :_K]
        r = lax.broadcasted_iota(jnp.int32, (_K, _K), 0)
        c = lax.broadcasted_iota(jnp.int32, (_K, _K), 1)
        eye = (r == c).astype(jnp.float32)
        cut = jnp.sum(adj_new * eye, keepdims=True)
        vol = jnp.sum(acc[:_K, _K:_K + 1], keepdims=True)
        mc_ref[...] = -cut / (vol + 1e-09)
        ss = ss_ref[:_K, :_K]
        or_ref[...] = jnp.sqrt(jnp.sum((ss - eye) ** 2, keepdims=True))
        a_mask = (adj_new > 0).astype(jnp.float32)
        zk = z_ref[:_K, :]
        x1 = _explain(zk, a_mask, e1w_top[...], e1w_bot[...], e1m1b[...],
                      e1m2w[...], e1m2b[...], e1lw[...], e1lb[...])
        x2 = _explain(x1, a_mask, e2w_top[...], e2w_bot[...], e2m1b[...],
                      e2m2w[...], e2m2b[...], e2lw[...], e2lb[...])
        out_ref[...] = (jnp.dot(x2, lw_ref[...],
                                preferred_element_type=jnp.float32)
                        + lb_ref[...])


def kernel(x, edge_index, assign_W, assign_b, proj_W, proj_b,
           e1_m1_W, e1_m1_b, e1_m2_W, e1_m2_b, e1_lin_W, e1_lin_b,
           e2_m1_W, e2_m1_b, e2_m2_W, e2_m2_b, e2_lin_W, e2_lin_b,
           lin_W, lin_b):
    n, d = x.shape
    e = edge_index.shape[1]
    blk = 1000
    n_blk = n // blk

    aw = jnp.pad(assign_W, ((0, 0), (0, 128 - _K)))
    ab = jnp.pad(assign_b, (0, 128 - _K), constant_values=-1e9).reshape(1, 128)
    pb = proj_b.reshape(1, d)

    s32, z32 = pl.pallas_call(
        _pass_a,
        grid=(n_blk,),
        in_specs=[
            pl.BlockSpec((blk, d), lambda i: (i, 0)),
            pl.BlockSpec((d, 128), lambda i: (0, 0)),
            pl.BlockSpec((1, 128), lambda i: (0, 0)),
            pl.BlockSpec((d, d), lambda i: (0, 0)),
            pl.BlockSpec((1, d), lambda i: (0, 0)),
        ],
        out_specs=[
            pl.BlockSpec((blk, 128), lambda i: (i, 0)),
            pl.BlockSpec((_KP, d), lambda i: (0, 0)),
        ],
        out_shape=[
            jax.ShapeDtypeStruct((n, 128), jnp.float32),
            jax.ShapeDtypeStruct((_KP, d), jnp.float32),
        ],
    )(x, aw, ab, proj_W, pb)

    ngrp = e // _LANE
    assert e % _LANE == 0
    e3 = edge_index.reshape(2, ngrp, _LANE).transpose(1, 0, 2)
    zeros = jnp.zeros((_NPAD, _KP), jnp.float32)

    parts = _make_seg_sum(ngrp)(s32, e3, zeros)

    e1m2w = e1_m2_W.reshape(1, -1)
    e2m2w = e2_m2_W.reshape(1, -1)

    def whole(shape):
        return pl.BlockSpec(shape, lambda i: tuple(0 for _ in shape))

    o = lin_W.shape[1]
    out, mc, ortho = pl.pallas_call(
        _pass_c,
        grid=(n_blk,),
        in_specs=[
            pl.BlockSpec((blk, 128), lambda i: (i, 0)),
            pl.BlockSpec((2, blk, 128), lambda i: (0, i, 0)),
            whole((_KP, d)),
            whole((d, d)), whole((d, d)), whole((1, d)),
            whole((1, d)), whole((1, 1)), whole((d, d)), whole((1, d)),
            whole((d, d)), whole((d, d)), whole((1, d)),
            whole((1, d)), whole((1, 1)), whole((d, d)), whole((1, d)),
            whole((d, o)), whole((1, o)),
        ],
        out_specs=[
            whole((_K, o)),
            whole((1, 1)),
            whole((1, 1)),
        ],
        out_shape=[
            jax.ShapeDtypeStruct((_K, o), jnp.float32),
            jax.ShapeDtypeStruct((1, 1), jnp.float32),
            jax.ShapeDtypeStruct((1, 1), jnp.float32),
        ],
        scratch_shapes=[
            pltpu.VMEM((_KP, _KP), jnp.float32),
            pltpu.VMEM((_KP, _KP), jnp.float32),
        ],
    )(s32, parts, z32,
      e1_m1_W[:d], e1_m1_W[d:], e1_m1_b.reshape(1, d),
      e1m2w, e1_m2_b.reshape(1, 1), e1_lin_W, e1_lin_b.reshape(1, -1),
      e2_m1_W[:d], e2_m1_W[d:], e2_m1_b.reshape(1, d),
      e2m2w, e2_m2_b.reshape(1, 1), e2_lin_W, e2_lin_b.reshape(1, -1),
      lin_W, lin_b.reshape(1, -1))

    return (out, mc.reshape(()), ortho.reshape(()), z32[:_K], s32[:, :_K])


# speedup vs baseline: 1.3410x; 1.0800x over previous
import functools

import jax
import jax.numpy as jnp
from jax import lax
from jax.experimental import pallas as pl
from jax.experimental.pallas import tpu as pltpu
from jax.experimental.pallas import tpu_sc as plsc

_N, _D, _K, _KP = 10000, 128, 30, 32
_NW = 32
_LANE = 128
_RPT = 320
_NPAD = 10240


def _pass_a(x_ref, aw_ref, ab_ref, pw_ref, pb_ref, s32_ref, z_ref):
    i = pl.program_id(0)
    xb = x_ref[...]
    logits = jnp.dot(xb, aw_ref[...], preferred_element_type=jnp.float32)
    logits = logits + ab_ref[...]
    m = jnp.max(logits, axis=-1, keepdims=True)
    e = jnp.exp(logits - m)
    s = e / jnp.sum(e, axis=-1, keepdims=True)
    colid = lax.broadcasted_iota(jnp.int32, s.shape, 1)
    s32 = jnp.where(colid == _K, 1.0, s)
    s32_ref[...] = s32
    xp = jnp.dot(xb, pw_ref[...], preferred_element_type=jnp.float32)
    xp = xp + pb_ref[...]
    zp = lax.dot_general(s32[:, :_KP], xp, (((0,), (0,)), ((), ())),
                         preferred_element_type=jnp.float32)

    @pl.when(i == 0)
    def _():
        z_ref[...] = zp

    @pl.when(i > 0)
    def _():
        z_ref[...] += zp


def _make_seg_sum(ngrp):
    mesh = plsc.VectorSubcoreMesh(core_axis_name="c", subcore_axis_name="s")

    depth = 8
    nbuf = 2 * depth
    q, rem = divmod(ngrp, _NW)
    assert q >= nbuf

    @functools.partial(
        pl.kernel,
        out_type=jax.ShapeDtypeStruct((2, _NPAD, 128), jnp.float32),
        mesh=mesh,
        scratch_types=[
            pltpu.VMEM((q + 1, _LANE), jnp.int32),
            pltpu.VMEM((q + 1, _LANE), jnp.int32),
            pltpu.VMEM((nbuf, _LANE, _KP), jnp.float32),
            pltpu.VMEM_SHARED((_NPAD, _KP), jnp.float32),
            pltpu.VMEM_SHARED((_N, _KP), jnp.float32),
            pltpu.SemaphoreType.DMA,
            pltpu.SemaphoreType.DMA,
        ],
        compiler_params=pltpu.CompilerParams(use_tc_tiling_on_sc=False),
    )
    def seg_sum(s32_hbm, e3_hbm, zeros_hbm, out_hbm,
                colv, rowv, rowsv, shared, table, gsem, ssem):
        cid = lax.axis_index("c")
        sid = lax.axis_index("s")
        wid = sid * 2 + cid
        base = q * wid + jnp.minimum(wid, rem)
        nch = q + jnp.where(wid < rem, 1, 0)
        pltpu.sync_copy(zeros_hbm.at[pl.ds(sid * 2 * _RPT, 2 * _RPT)],
                        shared.at[pl.ds(sid * 2 * _RPT, 2 * _RPT)])
        pltpu.sync_copy(
            s32_hbm.at[pl.ds(sid * (_N // 16), _N // 16), pl.ds(0, _KP)],
            table.at[pl.ds(sid * (_N // 16), _N // 16)])
        pltpu.sync_copy(e3_hbm.at[pl.ds(base, q), 1], colv.at[pl.ds(0, q)])
        pltpu.sync_copy(e3_hbm.at[pl.ds(base, q), 0], rowv.at[pl.ds(0, q)])

        @pl.when(wid < rem)
        def _():
            pltpu.sync_copy(e3_hbm.at[pl.ds(base + q, 1), 1],
                            colv.at[pl.ds(q, 1)])
            pltpu.sync_copy(e3_hbm.at[pl.ds(base + q, 1), 0],
                            rowv.at[pl.ds(q, 1)])

        plsc.subcore_barrier()

        def buf(j):
            return rowsv.at[lax.rem(j, nbuf)]

        def gstart(j):
            pltpu.async_copy(table.at[colv.at[j]], buf(j), gsem)

        def gwait(j):
            pltpu.make_async_copy(table.at[colv.at[j]], buf(j), gsem).wait()

        def sstart(j):
            pltpu.async_copy(buf(j), shared.at[rowv.at[j]], ssem, add=True)

        def swait(j):
            pltpu.make_async_copy(buf(j), shared.at[rowv.at[j]], ssem).wait()

        for j in range(depth):
            gstart(j)
        for j in range(depth):
            gstart(j + depth)
            gwait(j)
            sstart(j)

        def body(j, carry):
            swait(j - depth)
            gstart(j + depth)
            gwait(j)
            sstart(j)
            return carry

        lax.fori_loop(depth, nch - depth, body, 0)

        def tail(j, carry):
            swait(j - depth)
            gwait(j)
            sstart(j)
            return carry

        lax.fori_loop(nch - depth, nch, tail, 0)

        def drain(j, carry):
            swait(j)
            return carry

        lax.fori_loop(nch - depth, nch, drain, 0)
        plsc.subcore_barrier()
        pltpu.sync_copy(
            shared.at[pl.ds(sid * 2 * _RPT, 2 * _RPT)],
            out_hbm.at[cid, pl.ds(sid * 2 * _RPT, 2 * _RPT), pl.ds(0, _KP)])

    return seg_sum


def _explain(xk, a_mask, w_top, w_bot, m1_b, m2_w, m2_b, lin_w, lin_b):
    u = jnp.dot(xk, w_top, preferred_element_type=jnp.float32)
    v = jnp.dot(xk, w_bot, preferred_element_type=jnp.float32)
    h = jnp.maximum(u[:, None, :] + v[None, :, :] + m1_b, 0.0)
    logit = jnp.sum(h * m2_w, axis=-1) + m2_b[...]
    mask = (1.0 / (1.0 + jnp.exp(-logit))) * a_mask
    wsum = jnp.sum(mask, axis=1, keepdims=True)
    w = mask / (wsum + 1e-09)
    agg = jnp.dot(w, xk, preferred_element_type=jnp.float32)
    deg = jnp.sum(a_mask, axis=1, keepdims=True)
    emb = jnp.where(deg > 0, 0.5 * xk + 0.5 * agg, xk)
    out = jnp.dot(emb, lin_w, preferred_element_type=jnp.float32) + lin_b
    return jnp.maximum(out, 0.0)


def _pass_c(s32_ref, adj_ref, z_ref,
            e1w_top, e1w_bot, e1m1b, e1m2w, e1m2b, e1lw, e1lb,
            e2w_top, e2w_bot, e2m1b, e2m2w, e2m2b, e2lw, e2lb,
            lw_ref, lb_ref,
            out_ref, mc_ref, or_ref,
            acc_ref, ss_ref):
    i = pl.program_id(0)
    n_blk = pl.num_programs(0)
    s32 = s32_ref[...][:, :_KP]
    adj = adj_ref[0, :, :_KP] + adj_ref[1, :, :_KP]
    part = lax.dot_general(s32, adj, (((0,), (0,)), ((), ())),
                           preferred_element_type=jnp.float32)
    ssp = lax.dot_general(s32, s32, (((0,), (0,)), ((), ())),
                          preferred_element_type=jnp.float32)

    @pl.when(i == 0)
    def _():
        acc_ref[...] = part
        ss_ref[...] = ssp

    @pl.when(i > 0)
    def _():
        acc_ref[...] += part
        ss_ref[...] += ssp

    @pl.when(i == n_blk - 1)
    def _():
        acc = acc_ref[...]
        adj_new = acc[:_K, :_K]
        r = lax.broadcasted_iota(jnp.int32, (_K, _K), 0)
        c = lax.broadcasted_iota(jnp.int32, (_K, _K), 1)
        eye = (r == c).astype(jnp.float32)
        cut = jnp.sum(adj_new * eye, keepdims=True)
        vol = jnp.sum(acc[:_K, _K:_K + 1], keepdims=True)
        mc_ref[...] = -cut / (vol + 1e-09)
        ss = ss_ref[:_K, :_K]
        or_ref[...] = jnp.sqrt(jnp.sum((ss - eye) ** 2, keepdims=True))
        a_mask = (adj_new > 0).astype(jnp.float32)
        zk = z_ref[:_K, :]
        x1 = _explain(zk, a_mask, e1w_top[...], e1w_bot[...], e1m1b[...],
                      e1m2w[...], e1m2b[...], e1lw[...], e1lb[...])
        x2 = _explain(x1, a_mask, e2w_top[...], e2w_bot[...], e2m1b[...],
                      e2m2w[...], e2m2b[...], e2lw[...], e2lb[...])
        out_ref[...] = (jnp.dot(x2, lw_ref[...],
                                preferred_element_type=jnp.float32)
                        + lb_ref[...])


def kernel(x, edge_index, assign_W, assign_b, proj_W, proj_b,
           e1_m1_W, e1_m1_b, e1_m2_W, e1_m2_b, e1_lin_W, e1_lin_b,
           e2_m1_W, e2_m1_b, e2_m2_W, e2_m2_b, e2_lin_W, e2_lin_b,
           lin_W, lin_b):
    n, d = x.shape
    e = edge_index.shape[1]
    blk = 2000
    n_blk = n // blk

    aw = jnp.pad(assign_W, ((0, 0), (0, 128 - _K)))
    ab = jnp.pad(assign_b, (0, 128 - _K), constant_values=-1e9).reshape(1, 128)
    pb = proj_b.reshape(1, d)

    s32, z32 = pl.pallas_call(
        _pass_a,
        grid=(n_blk,),
        in_specs=[
            pl.BlockSpec((blk, d), lambda i: (i, 0)),
            pl.BlockSpec((d, 128), lambda i: (0, 0)),
            pl.BlockSpec((1, 128), lambda i: (0, 0)),
            pl.BlockSpec((d, d), lambda i: (0, 0)),
            pl.BlockSpec((1, d), lambda i: (0, 0)),
        ],
        out_specs=[
            pl.BlockSpec((blk, 128), lambda i: (i, 0)),
            pl.BlockSpec((_KP, d), lambda i: (0, 0)),
        ],
        out_shape=[
            jax.ShapeDtypeStruct((n, 128), jnp.float32),
            jax.ShapeDtypeStruct((_KP, d), jnp.float32),
        ],
    )(x, aw, ab, proj_W, pb)

    ngrp = e // _LANE
    assert e % _LANE == 0
    e3 = edge_index.reshape(2, ngrp, _LANE).transpose(1, 0, 2)
    zeros = jnp.zeros((_NPAD, _KP), jnp.float32)

    parts = _make_seg_sum(ngrp)(s32, e3, zeros)

    e1m2w = e1_m2_W.reshape(1, -1)
    e2m2w = e2_m2_W.reshape(1, -1)

    def whole(shape):
        return pl.BlockSpec(shape, lambda i: tuple(0 for _ in shape))

    o = lin_W.shape[1]
    out, mc, ortho = pl.pallas_call(
        _pass_c,
        grid=(n_blk,),
        in_specs=[
            pl.BlockSpec((blk, 128), lambda i: (i, 0)),
            pl.BlockSpec((2, blk, 128), lambda i: (0, i, 0)),
            whole((_KP, d)),
            whole((d, d)), whole((d, d)), whole((1, d)),
            whole((1, d)), whole((1, 1)), whole((d, d)), whole((1, d)),
            whole((d, d)), whole((d, d)), whole((1, d)),
            whole((1, d)), whole((1, 1)), whole((d, d)), whole((1, d)),
            whole((d, o)), whole((1, o)),
        ],
        out_specs=[
            whole((_K, o)),
            whole((1, 1)),
            whole((1, 1)),
        ],
        out_shape=[
            jax.ShapeDtypeStruct((_K, o), jnp.float32),
            jax.ShapeDtypeStruct((1, 1), jnp.float32),
            jax.ShapeDtypeStruct((1, 1), jnp.float32),
        ],
        scratch_shapes=[
            pltpu.VMEM((_KP, _KP), jnp.float32),
            pltpu.VMEM((_KP, _KP), jnp.float32),
        ],
    )(s32, parts, z32,
      e1_m1_W[:d], e1_m1_W[d:], e1_m1_b.reshape(1, d),
      e1m2w, e1_m2_b.reshape(1, 1), e1_lin_W, e1_lin_b.reshape(1, -1),
      e2_m1_W[:d], e2_m1_W[d:], e2_m1_b.reshape(1, d),
      e2m2w, e2_m2_b.reshape(1, 1), e2_lin_W, e2_lin_b.reshape(1, -1),
      lin_W, lin_b.reshape(1, -1))

    return (out, mc.reshape(()), ortho.reshape(()), z32[:_K], s32[:, :_K])
